# Initial kernel scaffold; baseline (speedup 1.0000x reference)
import jax
import jax.numpy as jnp
from jax.experimental import pallas as pl


def kernel(items_emb, items, items_emb_grad):
    raise NotImplementedError("write your pallas kernel here")


# speedup vs baseline: 2.1016x; 2.1016x over previous
import jax
import jax.numpy as jnp
from jax import lax
from jax.experimental import pallas as pl
from jax.experimental.pallas import tpu as pltpu
from jax.experimental.pallas import tpu_sc as plsc

M_ITEM = 1_000_000
DIM = 16
LR = 0.01

LANES = 16
R = 100_000
P = M_ITEM // R
PASSES_PER_CORE = P // 2
TILE_ROWS = R // 16
SUP = 1024


def _vec_at(vec, i):
    lane = lax.broadcasted_iota(jnp.int32, (LANES,), 0)
    return jnp.sum(jnp.where(lane == i, vec, 0))


def _body(emb, items, grads, bounds, out, bounds_v, kv, gv, idx2, shared):
    c = lax.axis_index("c")
    sid = lax.axis_index("s")
    pltpu.sync_copy(bounds, bounds_v)
    bv = bounds_v[...]

    def pass_body(tt, carry):
        p = 2 * tt + c
        base = p * R
        lo = _vec_at(bv, p)
        hi = _vec_at(bv, p + 1)
        lo_a = (lo // SUP) * SUP
        nsup = (hi - lo_a + (SUP - 1)) // SUP

        pltpu.sync_copy(emb.at[pl.ds(base + sid * TILE_ROWS, TILE_ROWS)],
                        shared.at[pl.ds(sid * TILE_ROWS, TILE_ROWS)])
        plsc.subcore_barrier()

        my_sup = (nsup - sid + 15) // 16

        def sup_body(j, carry2):
            s = lo_a + (sid + j * 16) * SUP
            pltpu.sync_copy(items.at[pl.ds(s, SUP)], kv)
            pltpu.sync_copy(grads.at[pl.ds(s, SUP)], gv)
            for q in range(SUP // LANES):
                keys = kv[pl.ds(q * LANES, LANES)]
                g = s + q * LANES + lax.broadcasted_iota(jnp.int32, (LANES,), 0)
                valid = (g >= lo) & (g < hi)
                idx2[q // 8, pl.ds((q % 8) * LANES, LANES)] = (
                    jnp.where(valid, keys - base, R))

            def scale(r, carry3):
                for k in range(4):
                    row = r * 4 + k
                    gv[row] = gv[row] * (-LR)
                return carry3
            lax.fori_loop(0, SUP // 4, scale, 0)

            for b in range(SUP // 128):
                pltpu.sync_copy(gv.at[pl.ds(b * 128, 128)],
                                shared.at[idx2.at[b]], add=True)
            return carry2
        lax.fori_loop(0, my_sup, sup_body, 0)
        plsc.subcore_barrier()

        pltpu.sync_copy(shared.at[pl.ds(sid * TILE_ROWS, TILE_ROWS)],
                        out.at[pl.ds(base + sid * TILE_ROWS, TILE_ROWS)])
        return carry

    lax.fori_loop(0, PASSES_PER_CORE, pass_body, 0)


def kernel(items_emb, items, items_emb_grad):
    items = items.astype(jnp.int32)
    edges = jnp.arange(0, M_ITEM + 1, R, dtype=jnp.int32)
    bounds = jnp.searchsorted(items, edges, side="left").astype(jnp.int32)
    bounds = jnp.pad(bounds, (0, LANES - bounds.shape[0]))

    mesh = plsc.VectorSubcoreMesh(core_axis_name="c", subcore_axis_name="s")
    run = pl.kernel(
        _body,
        out_type=jax.ShapeDtypeStruct((M_ITEM, DIM), jnp.float32),
        mesh=mesh,
        scratch_types=[
            pltpu.VMEM((LANES,), jnp.int32),
            pltpu.VMEM((SUP,), jnp.int32),
            pltpu.VMEM((SUP, DIM), jnp.float32),
            pltpu.VMEM((SUP // 128, 128), jnp.int32),
            pltpu.VMEM_SHARED((R + 1, DIM), jnp.float32),
        ],
        compiler_params=pltpu.CompilerParams(
            use_tc_tiling_on_sc=False, needs_layout_passes=False),
    )
    return run(items_emb, items, items_emb_grad, bounds)
